# baseline (speedup 1.0000x reference)
import functools
import math

import jax
import jax.numpy as jnp
from jax import lax
from jax.experimental import pallas as pl
from jax.experimental.pallas import tpu as pltpu
from jax.experimental.pallas import tpu_sc as plsc

N = 10000
C = 128
E = 320000
S = 64
H = 16
DH = C // H

NC = 2
NS = 16
NW = NC * NS
CH = 128
CHUNKS = 2560
EPAD = CHUNKS * CH
CPT = CHUNKS // NW
NP = CPT // 2
RBASE = 624
RPS = 640


def _sc_segment_sum(xf, row1, col1, val1):
    mesh = plsc.VectorSubcoreMesh(core_axis_name="c", subcore_axis_name="s")

    @functools.partial(
        pl.kernel,
        out_type=jax.ShapeDtypeStruct((NC, N, C), jnp.float32),
        mesh=mesh,
        scratch_types=[
            pltpu.VMEM((CH,), jnp.int32),
            pltpu.VMEM((CH,), jnp.int32),
            pltpu.VMEM((CH,), jnp.float32),
            pltpu.VMEM((CH,), jnp.int32),
            pltpu.VMEM((CH,), jnp.int32),
            pltpu.VMEM((CH,), jnp.float32),
            pltpu.VMEM((CH, C), jnp.float32),
            pltpu.VMEM((CH, C), jnp.float32),
            pltpu.VMEM_SHARED((N, C), jnp.float32),
            pltpu.SemaphoreType.DMA,
            pltpu.SemaphoreType.DMA,
            pltpu.SemaphoreType.DMA,
            pltpu.SemaphoreType.DMA,
            pltpu.SemaphoreType.DMA,
        ],
    )
    def seg_sum(x_hbm, row_hbm, col_hbm, val_hbm, out_hbm,
                colc0, rowc0, valc0, colc1, rowc1, valc1, rows0, rows1, acc,
                isem0, isem1, gsem0, gsem1, ssem0):
        cid = lax.axis_index("c")
        sid = lax.axis_index("s")
        wid = cid * NS + sid
        ebase = wid * CPT * CH

        def zrow(r, carry):
            for q in range(8):
                rows0[r, pl.ds(q * 16, 16)] = jnp.zeros((16,), jnp.float32)
            return carry
        lax.fori_loop(0, CH, zrow, 0)
        base_r = sid * RBASE
        for k in range(RPS // CH):
            pltpu.sync_copy(rows0, acc.at[pl.ds(base_r + k * CH, CH)])
        plsc.subcore_barrier()

        _dnums = lax.GatherDimensionNumbers(
            offset_dims=(), collapsed_slice_dims=(0,), start_index_map=(0,))

        def _splat(vec, j):
            return lax.gather(vec, jnp.full((16, 1), j, jnp.int32), _dnums,
                              (1,), mode=lax.GatherScatterMode.PROMISE_IN_BOUNDS)

        def scale_rows(buf, valc):
            @plsc.parallel_loop(0, CH // 16, 1, unroll=2)
            def _(g):
                vv = valc[pl.ds(g * 16, 16)]
                for j in range(16):
                    r = g * 16 + j
                    vs = _splat(vv, j)
                    for q in range(8):
                        buf[r, pl.ds(q * 16, 16)] = buf[r, pl.ds(q * 16, 16)] * vs

        def fetch_idx(c, colc, rowc, valc, sem):
            off = ebase + c * CH
            pltpu.async_copy(col_hbm.at[pl.ds(off, CH)], colc, sem)
            pltpu.async_copy(row_hbm.at[pl.ds(off, CH)], rowc, sem)
            pltpu.async_copy(val_hbm.at[pl.ds(off, CH)], valc, sem)

        def wait_idx(colc, rowc, valc, sem):
            pltpu.make_async_copy(col_hbm.at[pl.ds(0, CH)], colc, sem).wait()
            pltpu.make_async_copy(row_hbm.at[pl.ds(0, CH)], rowc, sem).wait()
            pltpu.make_async_copy(val_hbm.at[pl.ds(0, CH)], valc, sem).wait()

        def wait_gather(buf, sem):
            pltpu.make_async_copy(x_hbm.at[pl.ds(0, CH)], buf, sem).wait()

        fetch_idx(0, colc0, rowc0, valc0, isem0)
        fetch_idx(1, colc1, rowc1, valc1, isem1)
        wait_idx(colc0, rowc0, valc0, isem0)
        pltpu.async_copy(x_hbm.at[colc0], rows0, gsem0)

        def pair(i2, carry):
            c0 = 2 * i2

            wait_idx(colc1, rowc1, valc1, isem1)
            pltpu.async_copy(x_hbm.at[colc1], rows1, gsem1)

            wait_gather(rows0, gsem0)
            scale_rows(rows0, valc0)
            s0 = pltpu.async_copy(rows0, acc.at[rowc0], ssem0, add=True)

            wait_gather(rows1, gsem1)
            scale_rows(rows1, valc1)
            s0.wait()
            @pl.when(i2 < NP - 1)
            def _():
                fetch_idx(c0 + 2, colc0, rowc0, valc0, isem0)
            pltpu.sync_copy(rows1, acc.at[rowc1], add=True)

            @pl.when(i2 < NP - 1)
            def _():
                wait_idx(colc0, rowc0, valc0, isem0)
                pltpu.async_copy(x_hbm.at[colc0], rows0, gsem0)
                fetch_idx(c0 + 3, colc1, rowc1, valc1, isem1)
            return carry
        lax.fori_loop(0, NP, pair, 0)

        plsc.subcore_barrier()
        pltpu.sync_copy(acc.at[pl.ds(base_r, RPS)],
                        out_hbm.at[cid, pl.ds(base_r, RPS)])

    return seg_sum(xf, row1, col1, val1)


BL = 1000
_SQRT_HALF = 1.0 / math.sqrt(2.0)


def _passA_body(x_ref, g0_ref, g1_ref, was_ref, bas_ref, wag_ref,
                lng_ref, lnb_ref, wsp_ref, bsp_ref,
                w_ref, sacc_ref, wsum_ref):
    i = pl.program_id(0)
    xb = x_ref[...]
    xg = g0_ref[...] + g1_ref[...]
    a = (lax.dot_general(xb, was_ref[...], (((1,), (1,)), ((), ())),
                         preferred_element_type=jnp.float32)
         + lax.dot_general(xg, wag_ref[...], (((1,), (1,)), ((), ())),
                           preferred_element_type=jnp.float32)
         + bas_ref[...])
    mu = jnp.mean(a, axis=1, keepdims=True)
    d = a - mu
    var = jnp.mean(d * d, axis=1, keepdims=True)
    an = d * lax.rsqrt(var + 1e-5) * lng_ref[...] + lnb_ref[...]
    ge = 0.5 * an * (1.0 + lax.erf(an * _SQRT_HALF))
    logits = lax.dot_general(ge, wsp_ref[...], (((1,), (1,)), ((), ())),
                             preferred_element_type=jnp.float32) + bsp_ref[...]
    m = jnp.max(logits, axis=1, keepdims=True)
    e = jnp.exp(logits - m)
    w = e / jnp.sum(e, axis=1, keepdims=True)
    w_ref[...] = w

    @pl.when(i == 0)
    def _():
        sacc_ref[...] = jnp.zeros_like(sacc_ref)
        wsum_ref[...] = jnp.zeros_like(wsum_ref)

    sacc_ref[...] += lax.dot_general(w, xb, (((0,), (0,)), ((), ())),
                                     preferred_element_type=jnp.float32)
    wsum_ref[...] += jnp.sum(w, axis=0, keepdims=True)


def _passA(xf, xg0, xg1, W_as, b_as, W_ag, ln_g, ln_b, W_sp, b_sp,
           interpret=False):
    grid = (N // BL,)
    row_spec = pl.BlockSpec((BL, C), lambda i: (i, 0))
    full = lambda shape: pl.BlockSpec(shape, lambda i: (0, 0))
    return pl.pallas_call(
        _passA_body,
        grid=grid,
        in_specs=[row_spec, row_spec, row_spec,
                  full((C, C)), full((1, C)), full((C, C)),
                  full((1, C)), full((1, C)), full((S, C)), full((1, S))],
        out_specs=[pl.BlockSpec((BL, S), lambda i: (i, 0)),
                   full((S, C)), full((1, S))],
        out_shape=[jax.ShapeDtypeStruct((N, S), jnp.float32),
                   jax.ShapeDtypeStruct((S, C), jnp.float32),
                   jax.ShapeDtypeStruct((1, S), jnp.float32)],
        interpret=interpret,
    )(xf, xg0, xg1, W_as, b_as, W_ag, ln_g, ln_b, W_sp, b_sp)


def _passB_body(sacc_ref, wsum_ref, wq_ref, wk_ref, wv_ref,
                bq_ref, bk_ref, bv_ref, wo_ref, bo_ref, out_ref):
    ws = jnp.maximum(wsum_ref[...], 1e-8)
    s = sacc_ref[...] * (1.0 / ws)
    dims = (((1,), (1,)), ((), ()))
    q = lax.dot_general(s, wq_ref[...], dims,
                        preferred_element_type=jnp.float32) + bq_ref[...]
    k = lax.dot_general(s, wk_ref[...], dims,
                        preferred_element_type=jnp.float32) + bk_ref[...]
    v = lax.dot_general(s, wv_ref[...], dims,
                        preferred_element_type=jnp.float32) + bv_ref[...]
    colh = lax.broadcasted_iota(jnp.int32, (1, C), 1) // DH
    o = jnp.zeros((S, C), jnp.float32)
    scale = 1.0 / math.sqrt(DH)
    for h in range(H):
        mh = (colh == h).astype(jnp.float32)
        qh = q * mh
        sc = lax.dot_general(qh, k, dims,
                             preferred_element_type=jnp.float32) * scale
        m = jnp.max(sc, axis=1, keepdims=True)
        eh = jnp.exp(sc - m)
        at = eh / jnp.sum(eh, axis=1, keepdims=True)
        o = o + lax.dot_general(at, v * mh, (((1,), (0,)), ((), ())),
                                preferred_element_type=jnp.float32)
    out_ref[...] = lax.dot_general(o, wo_ref[...], dims,
                                   preferred_element_type=jnp.float32) + bo_ref[...]


def _passB(sacc, wsumT, Wq, Wk, Wv, bq, bk, bv, Wo, bo, interpret=False):
    return pl.pallas_call(
        _passB_body,
        out_shape=jax.ShapeDtypeStruct((S, C), jnp.float32),
        interpret=interpret,
    )(sacc, wsumT, Wq, Wk, Wv, bq, bk, bv, Wo, bo)


def _passC_body(w_ref, so_ref, out_ref):
    out_ref[...] = lax.dot_general(w_ref[...], so_ref[...],
                                   (((1,), (0,)), ((), ())),
                                   preferred_element_type=jnp.float32)


def _passC(weights, so, interpret=False):
    return pl.pallas_call(
        _passC_body,
        grid=(N // BL,),
        in_specs=[pl.BlockSpec((BL, S), lambda i: (i, 0)),
                  pl.BlockSpec((S, C), lambda i: (0, 0))],
        out_specs=pl.BlockSpec((BL, C), lambda i: (i, 0)),
        out_shape=jax.ShapeDtypeStruct((N, C), jnp.float32),
        interpret=interpret,
    )(weights, so)


def kernel(x, adj_indices, adj_values, W_as, b_as, W_ag, ln_g, ln_b,
           W_sp, b_sp, in_proj_w, in_proj_b, out_w, out_b):
    xf = x.reshape(N, C)
    pad = EPAD - E
    spread = (jnp.arange(pad, dtype=jnp.int32) * 37) % N
    row1 = jnp.concatenate([adj_indices[0].astype(jnp.int32), spread])
    col1 = jnp.concatenate([adj_indices[1].astype(jnp.int32), spread])
    val1 = jnp.concatenate([adj_values.astype(jnp.float32),
                            jnp.zeros((pad,), jnp.float32)])
    parts = _sc_segment_sum(xf, row1, col1, val1)

    weights, sacc, wsum = _passA(
        xf, parts[0], parts[1], W_as, b_as.reshape(1, C), W_ag,
        ln_g.reshape(1, C), ln_b.reshape(1, C), W_sp, b_sp.reshape(1, S))

    Wq, Wk, Wv = in_proj_w[:C], in_proj_w[C:2 * C], in_proj_w[2 * C:]
    bq = in_proj_b[:C].reshape(1, C)
    bk = in_proj_b[C:2 * C].reshape(1, C)
    bv = in_proj_b[2 * C:].reshape(1, C)

    so = _passB(sacc, wsum.reshape(S, 1), Wq, Wk, Wv, bq, bk, bv,
                out_w, out_b.reshape(1, C))
    out = _passC(weights, so)
    return out.reshape(1, N, C)


# speedup vs baseline: 1.0156x; 1.0156x over previous
import functools
import math

import jax
import jax.numpy as jnp
from jax import lax
from jax.experimental import pallas as pl
from jax.experimental.pallas import tpu as pltpu
from jax.experimental.pallas import tpu_sc as plsc

N = 10000
C = 128
E = 320000
S = 64
H = 16
DH = C // H

NC = 2
NS = 16
NW = NC * NS
CH = 128
CHUNKS = 2560
EPAD = CHUNKS * CH
CPT = CHUNKS // NW
NP = CPT // 2
RBASE = 624
RPS = 640


def _sc_segment_sum(xf, row1, col1, val1):
    mesh = plsc.VectorSubcoreMesh(core_axis_name="c", subcore_axis_name="s")

    @functools.partial(
        pl.kernel,
        out_type=jax.ShapeDtypeStruct((NC, N, C), jnp.float32),
        mesh=mesh,
        scratch_types=[
            pltpu.VMEM((CH,), jnp.int32),
            pltpu.VMEM((CH,), jnp.int32),
            pltpu.VMEM((CH,), jnp.float32),
            pltpu.VMEM((CH,), jnp.int32),
            pltpu.VMEM((CH,), jnp.int32),
            pltpu.VMEM((CH,), jnp.float32),
            pltpu.VMEM((CH,), jnp.int32),
            pltpu.VMEM((CH, C), jnp.float32),
            pltpu.VMEM((CH, C), jnp.float32),
            pltpu.VMEM_SHARED((N, C), jnp.float32),
            pltpu.SemaphoreType.DMA,
            pltpu.SemaphoreType.DMA,
            pltpu.SemaphoreType.DMA,
            pltpu.SemaphoreType.DMA,
            pltpu.SemaphoreType.DMA,
            pltpu.SemaphoreType.DMA,
        ],
    )
    def seg_sum(x_hbm, row_hbm, col_hbm, val_hbm, out_hbm,
                colc0, rowc0, valc0, colc1, rowc1, valc1, srow1,
                rows0, rows1, acc,
                isem0, isem1, gsem0, gsem1, ssem0, ssem1):
        cid = lax.axis_index("c")
        sid = lax.axis_index("s")
        wid = cid * NS + sid
        ebase = wid * CPT * CH

        def zrow(r, carry):
            for q in range(8):
                rows0[r, pl.ds(q * 16, 16)] = jnp.zeros((16,), jnp.float32)
            return carry
        lax.fori_loop(0, CH, zrow, 0)
        base_r = sid * RBASE
        for k in range(RPS // CH):
            pltpu.sync_copy(rows0, acc.at[pl.ds(base_r + k * CH, CH)])
        plsc.subcore_barrier()

        _dnums = lax.GatherDimensionNumbers(
            offset_dims=(), collapsed_slice_dims=(0,), start_index_map=(0,))

        def _splat(vec, j):
            return lax.gather(vec, jnp.full((16, 1), j, jnp.int32), _dnums,
                              (1,), mode=lax.GatherScatterMode.PROMISE_IN_BOUNDS)

        def scale_rows(buf, valc):
            @plsc.parallel_loop(0, CH // 16, 1, unroll=2)
            def _(g):
                vv = valc[pl.ds(g * 16, 16)]
                for j in range(16):
                    r = g * 16 + j
                    vs = _splat(vv, j)
                    for q in range(8):
                        buf[r, pl.ds(q * 16, 16)] = buf[r, pl.ds(q * 16, 16)] * vs

        def fetch_idx(c, colc, rowc, valc, sem):
            off = ebase + c * CH
            pltpu.async_copy(col_hbm.at[pl.ds(off, CH)], colc, sem)
            pltpu.async_copy(row_hbm.at[pl.ds(off, CH)], rowc, sem)
            pltpu.async_copy(val_hbm.at[pl.ds(off, CH)], valc, sem)

        def wait_idx(colc, rowc, valc, sem):
            pltpu.make_async_copy(col_hbm.at[pl.ds(0, CH)], colc, sem).wait()
            pltpu.make_async_copy(row_hbm.at[pl.ds(0, CH)], rowc, sem).wait()
            pltpu.make_async_copy(val_hbm.at[pl.ds(0, CH)], valc, sem).wait()

        def wait_gather(buf, sem):
            pltpu.make_async_copy(x_hbm.at[pl.ds(0, CH)], buf, sem).wait()

        fetch_idx(0, colc0, rowc0, valc0, isem0)
        fetch_idx(1, colc1, rowc1, valc1, isem1)
        wait_idx(colc0, rowc0, valc0, isem0)
        pltpu.async_copy(x_hbm.at[colc0], rows0, gsem0)

        def wait_scatter1():
            pltpu.make_async_copy(rows1, acc.at[pl.ds(0, CH)], ssem1).wait()

        def pair(i2, carry):
            c0 = 2 * i2

            wait_idx(colc1, rowc1, valc1, isem1)

            @pl.when(i2 > 0)
            def _():
                wait_scatter1()
            pltpu.async_copy(x_hbm.at[colc1], rows1, gsem1)

            wait_gather(rows0, gsem0)
            scale_rows(rows0, valc0)
            s0 = pltpu.async_copy(rows0, acc.at[rowc0], ssem0, add=True)

            wait_gather(rows1, gsem1)
            scale_rows(rows1, valc1)
            for t in range(CH // 16):
                srow1[pl.ds(t * 16, 16)] = rowc1[pl.ds(t * 16, 16)]
            pltpu.async_copy(rows1, acc.at[srow1], ssem1, add=True)
            s0.wait()

            @pl.when(i2 < NP - 1)
            def _():
                fetch_idx(c0 + 2, colc0, rowc0, valc0, isem0)
                wait_idx(colc0, rowc0, valc0, isem0)
                pltpu.async_copy(x_hbm.at[colc0], rows0, gsem0)
                fetch_idx(c0 + 3, colc1, rowc1, valc1, isem1)
            return carry
        lax.fori_loop(0, NP, pair, 0)
        wait_scatter1()

        plsc.subcore_barrier()
        pltpu.sync_copy(acc.at[pl.ds(base_r, RPS)],
                        out_hbm.at[cid, pl.ds(base_r, RPS)])

    return seg_sum(xf, row1, col1, val1)


BL = 1000
_SQRT_HALF = 1.0 / math.sqrt(2.0)


def _passA_body(x_ref, g0_ref, g1_ref, was_ref, bas_ref, wag_ref,
                lng_ref, lnb_ref, wsp_ref, bsp_ref,
                w_ref, sacc_ref, wsum_ref):
    i = pl.program_id(0)
    xb = x_ref[...]
    xg = g0_ref[...] + g1_ref[...]
    a = (lax.dot_general(xb, was_ref[...], (((1,), (1,)), ((), ())),
                         preferred_element_type=jnp.float32)
         + lax.dot_general(xg, wag_ref[...], (((1,), (1,)), ((), ())),
                           preferred_element_type=jnp.float32)
         + bas_ref[...])
    mu = jnp.mean(a, axis=1, keepdims=True)
    d = a - mu
    var = jnp.mean(d * d, axis=1, keepdims=True)
    an = d * lax.rsqrt(var + 1e-5) * lng_ref[...] + lnb_ref[...]
    ge = 0.5 * an * (1.0 + lax.erf(an * _SQRT_HALF))
    logits = lax.dot_general(ge, wsp_ref[...], (((1,), (1,)), ((), ())),
                             preferred_element_type=jnp.float32) + bsp_ref[...]
    m = jnp.max(logits, axis=1, keepdims=True)
    e = jnp.exp(logits - m)
    w = e / jnp.sum(e, axis=1, keepdims=True)
    w_ref[...] = w

    @pl.when(i == 0)
    def _():
        sacc_ref[...] = jnp.zeros_like(sacc_ref)
        wsum_ref[...] = jnp.zeros_like(wsum_ref)

    sacc_ref[...] += lax.dot_general(w, xb, (((0,), (0,)), ((), ())),
                                     preferred_element_type=jnp.float32)
    wsum_ref[...] += jnp.sum(w, axis=0, keepdims=True)


def _passA(xf, xg0, xg1, W_as, b_as, W_ag, ln_g, ln_b, W_sp, b_sp,
           interpret=False):
    grid = (N // BL,)
    row_spec = pl.BlockSpec((BL, C), lambda i: (i, 0))
    full = lambda shape: pl.BlockSpec(shape, lambda i: (0, 0))
    return pl.pallas_call(
        _passA_body,
        grid=grid,
        in_specs=[row_spec, row_spec, row_spec,
                  full((C, C)), full((1, C)), full((C, C)),
                  full((1, C)), full((1, C)), full((S, C)), full((1, S))],
        out_specs=[pl.BlockSpec((BL, S), lambda i: (i, 0)),
                   full((S, C)), full((1, S))],
        out_shape=[jax.ShapeDtypeStruct((N, S), jnp.float32),
                   jax.ShapeDtypeStruct((S, C), jnp.float32),
                   jax.ShapeDtypeStruct((1, S), jnp.float32)],
        interpret=interpret,
    )(xf, xg0, xg1, W_as, b_as, W_ag, ln_g, ln_b, W_sp, b_sp)


def _passB_body(sacc_ref, wsum_ref, wq_ref, wk_ref, wv_ref,
                bq_ref, bk_ref, bv_ref, wo_ref, bo_ref, out_ref):
    ws = jnp.maximum(wsum_ref[...], 1e-8)
    s = sacc_ref[...] * (1.0 / ws)
    dims = (((1,), (1,)), ((), ()))
    q = lax.dot_general(s, wq_ref[...], dims,
                        preferred_element_type=jnp.float32) + bq_ref[...]
    k = lax.dot_general(s, wk_ref[...], dims,
                        preferred_element_type=jnp.float32) + bk_ref[...]
    v = lax.dot_general(s, wv_ref[...], dims,
                        preferred_element_type=jnp.float32) + bv_ref[...]
    colh = lax.broadcasted_iota(jnp.int32, (1, C), 1) // DH
    o = jnp.zeros((S, C), jnp.float32)
    scale = 1.0 / math.sqrt(DH)
    for h in range(H):
        mh = (colh == h).astype(jnp.float32)
        qh = q * mh
        sc = lax.dot_general(qh, k, dims,
                             preferred_element_type=jnp.float32) * scale
        m = jnp.max(sc, axis=1, keepdims=True)
        eh = jnp.exp(sc - m)
        at = eh / jnp.sum(eh, axis=1, keepdims=True)
        o = o + lax.dot_general(at, v * mh, (((1,), (0,)), ((), ())),
                                preferred_element_type=jnp.float32)
    out_ref[...] = lax.dot_general(o, wo_ref[...], dims,
                                   preferred_element_type=jnp.float32) + bo_ref[...]


def _passB(sacc, wsumT, Wq, Wk, Wv, bq, bk, bv, Wo, bo, interpret=False):
    return pl.pallas_call(
        _passB_body,
        out_shape=jax.ShapeDtypeStruct((S, C), jnp.float32),
        interpret=interpret,
    )(sacc, wsumT, Wq, Wk, Wv, bq, bk, bv, Wo, bo)


def _passC_body(w_ref, so_ref, out_ref):
    out_ref[...] = lax.dot_general(w_ref[...], so_ref[...],
                                   (((1,), (0,)), ((), ())),
                                   preferred_element_type=jnp.float32)


def _passC(weights, so, interpret=False):
    return pl.pallas_call(
        _passC_body,
        grid=(N // BL,),
        in_specs=[pl.BlockSpec((BL, S), lambda i: (i, 0)),
                  pl.BlockSpec((S, C), lambda i: (0, 0))],
        out_specs=pl.BlockSpec((BL, C), lambda i: (i, 0)),
        out_shape=jax.ShapeDtypeStruct((N, C), jnp.float32),
        interpret=interpret,
    )(weights, so)


def kernel(x, adj_indices, adj_values, W_as, b_as, W_ag, ln_g, ln_b,
           W_sp, b_sp, in_proj_w, in_proj_b, out_w, out_b):
    xf = x.reshape(N, C)
    pad = EPAD - E
    spread = (jnp.arange(pad, dtype=jnp.int32) * 37) % N
    row1 = jnp.concatenate([adj_indices[0].astype(jnp.int32), spread])
    col1 = jnp.concatenate([adj_indices[1].astype(jnp.int32), spread])
    val1 = jnp.concatenate([adj_values.astype(jnp.float32),
                            jnp.zeros((pad,), jnp.float32)])
    parts = _sc_segment_sum(xf, row1, col1, val1)

    weights, sacc, wsum = _passA(
        xf, parts[0], parts[1], W_as, b_as.reshape(1, C), W_ag,
        ln_g.reshape(1, C), ln_b.reshape(1, C), W_sp, b_sp.reshape(1, S))

    Wq, Wk, Wv = in_proj_w[:C], in_proj_w[C:2 * C], in_proj_w[2 * C:]
    bq = in_proj_b[:C].reshape(1, C)
    bk = in_proj_b[C:2 * C].reshape(1, C)
    bv = in_proj_b[2 * C:].reshape(1, C)

    so = _passB(sacc, wsum.reshape(S, 1), Wq, Wk, Wv, bq, bk, bv,
                out_w, out_b.reshape(1, C))
    out = _passC(weights, so)
    return out.reshape(1, N, C)
